# baseline (speedup 1.0000x reference)
import functools

import jax
import jax.numpy as jnp
from jax import lax
from jax.experimental import pallas as pl
from jax.experimental.pallas import tpu as pltpu
from jax.experimental.pallas import tpu_sc as plsc

N_NODES = 10000
NFEAT = 128
HIDDEN = 64
NCLASS = 64
N_EDGES = 320000

NC = 2
NS = 16
NW = NC * NS
CH = 128
NCHUNK = 80
NBUF = 4
E_PER_TILE = NCHUNK * CH
E_PAD = NW * E_PER_TILE
ACC_ROWS = 10240
RPT = ACC_ROWS // NS
TPT = N_NODES // NS

_mesh = plsc.VectorSubcoreMesh(core_axis_name="c", subcore_axis_name="s")


def _sc_scatter_kernel(width):
  scratch = [
      pltpu.VMEM((NCHUNK, CH), jnp.int32),
      pltpu.VMEM((NCHUNK, CH), jnp.int32),
      pltpu.VMEM((CH, width), jnp.float32),
      pltpu.VMEM_SHARED((ACC_ROWS, width), jnp.float32),
      pltpu.SemaphoreType.DMA,
  ]
  if width != 16:
    for _ in range(NBUF - 1):
      scratch.append(pltpu.VMEM((CH, width), jnp.float32))
      scratch.append(pltpu.SemaphoreType.DMA)

  @functools.partial(
      pl.kernel,
      mesh=_mesh,
      out_type=jax.ShapeDtypeStruct((NC, ACC_ROWS, width), jnp.float32),
      scratch_types=scratch,
      compiler_params=pltpu.CompilerParams(use_tc_tiling_on_sc=False),
  )
  def k(table_hbm, src_hbm, dst_hbm, zeros_hbm, out_hbm,
        src_v, dst_v, rows_v, acc_sh, sem, *maybe_tab):
    cid = lax.axis_index("c")
    sid = lax.axis_index("s")
    wid = cid * NS + sid
    sl = pl.ds(sid * RPT, RPT)
    pltpu.sync_copy(zeros_hbm.at[sl], acc_sh.at[sl])
    pltpu.sync_copy(src_hbm.at[wid], src_v)
    pltpu.sync_copy(dst_hbm.at[wid], dst_v)
    if width == 16:
      pltpu.sync_copy(table_hbm, rows_v)
      plsc.subcore_barrier()

      @pl.loop(0, NCHUNK)
      def _(j):
        pltpu.sync_copy(rows_v, acc_sh.at[dst_v.at[j]], add=True)
    else:
      bufs = [rows_v] + list(maybe_tab[0::2])
      sems = [sem] + list(maybe_tab[1::2])
      plsc.subcore_barrier()

      for b in range(NBUF - 1):
        pltpu.async_copy(table_hbm.at[src_v.at[b]], bufs[b], sems[b])

      @pl.loop(0, NCHUNK // NBUF)
      def _(i):
        j = i * NBUF
        for b in range(NBUF):
          jb = j + b
          nxt = jb + NBUF - 1
          nb = (b + NBUF - 1) % NBUF

          @pl.when(nxt < NCHUNK)
          def _(nxt=nxt, nb=nb):
            pltpu.async_copy(table_hbm.at[src_v.at[nxt]], bufs[nb], sems[nb])

          pltpu.make_async_copy(
              table_hbm.at[src_v.at[jb]], bufs[b], sems[b]).wait()
          pltpu.sync_copy(bufs[b], acc_sh.at[dst_v.at[jb]], add=True)

    plsc.subcore_barrier()
    pltpu.sync_copy(acc_sh.at[sl], out_hbm.at[cid, sl])

  return k


_sc_degree = _sc_scatter_kernel(16)
_sc_aggregate = _sc_scatter_kernel(64)


def _tc_table1(deg_parts, x, w):
  def body(p_ref, x_ref, w_ref, dinv_ref, tab_ref):
    deg = p_ref[0, :N_NODES, 0:1] + p_ref[1, :N_NODES, 0:1] + 1.0
    dinv = lax.rsqrt(deg)
    dinv_ref[...] = dinv
    xw = jnp.dot(x_ref[...], w_ref[...],
                 preferred_element_type=jnp.float32,
                 precision=lax.Precision.HIGHEST)
    tab_ref[...] = xw * dinv

  return pl.pallas_call(
      body,
      out_shape=(
          jax.ShapeDtypeStruct((N_NODES, 1), jnp.float32),
          jax.ShapeDtypeStruct((N_NODES, HIDDEN), jnp.float32),
      ),
  )(deg_parts, x, w)


def _tc_mid(parts, tab1, dinv, b1, w2):
  def body(p_ref, t_ref, d_ref, b_ref, w_ref, o_ref):
    s = p_ref[0, :N_NODES] + p_ref[1, :N_NODES]
    dinv = d_ref[...]
    h = jnp.maximum((s + t_ref[...]) * dinv + b_ref[...], 0.0)
    o_ref[...] = jnp.dot(h, w_ref[...],
                         preferred_element_type=jnp.float32,
                         precision=lax.Precision.HIGHEST) * dinv

  return pl.pallas_call(
      body,
      out_shape=jax.ShapeDtypeStruct((N_NODES, NCLASS), jnp.float32),
  )(parts, tab1, dinv, b1, w2)


def _tc_out(parts, tab2, dinv, b2):
  def body(p_ref, t_ref, d_ref, b_ref, o_ref):
    s = p_ref[0, :N_NODES] + p_ref[1, :N_NODES]
    o_ref[...] = (s + t_ref[...]) * d_ref[...] + b_ref[...]

  return pl.pallas_call(
      body,
      out_shape=jax.ShapeDtypeStruct((N_NODES, NCLASS), jnp.float32),
  )(parts, tab2, dinv, b2)


def kernel(x, edge_index, W1, b1, W2, b2):
  src = edge_index[0].astype(jnp.int32)
  dst = edge_index[1].astype(jnp.int32)
  pad = E_PAD - N_EDGES
  src3 = jnp.concatenate([src, jnp.zeros((pad,), jnp.int32)]).reshape(
      NW, NCHUNK, CH)
  dst3 = jnp.concatenate([dst, jnp.full((pad,), N_NODES, jnp.int32)]).reshape(
      NW, NCHUNK, CH)

  zeros16 = jnp.zeros((ACC_ROWS, 16), jnp.float32)
  zeros64 = jnp.zeros((ACC_ROWS, 64), jnp.float32)
  ones16 = jnp.ones((CH, 16), jnp.float32)

  deg_parts = _sc_degree(ones16, src3, dst3, zeros16)
  dinv, tab1 = _tc_table1(deg_parts, x, W1)

  s1 = _sc_aggregate(tab1, src3, dst3, zeros64)
  tab2 = _tc_mid(s1, tab1, dinv, b1.reshape(1, HIDDEN), W2)

  s2 = _sc_aggregate(tab2, src3, dst3, zeros64)
  return _tc_out(s2, tab2, dinv, b2.reshape(1, NCLASS))


# speedup vs baseline: 2.2535x; 2.2535x over previous
import functools

import jax
import jax.numpy as jnp
from jax import lax
from jax.experimental import pallas as pl
from jax.experimental.pallas import tpu as pltpu
from jax.experimental.pallas import tpu_sc as plsc

N_NODES = 10000
NFEAT = 128
HIDDEN = 64
NCLASS = 64
N_EDGES = 320000

NC = 2
NS = 16
NW = NC * NS
CH = 128
NCHUNK = 80
NBUF = 2
E_PER_TILE = NCHUNK * CH
E_PAD = NW * E_PER_TILE
ACC_ROWS = 10240
RPT = ACC_ROWS // NS
TPT = N_NODES // NS

_mesh = plsc.VectorSubcoreMesh(core_axis_name="c", subcore_axis_name="s")


def _sc_scatter_kernel(width):
  scratch = [
      pltpu.VMEM((NCHUNK, CH), jnp.int32),
      pltpu.VMEM((NCHUNK, CH), jnp.int32),
      pltpu.VMEM((CH, width), jnp.float32),
      pltpu.VMEM_SHARED((ACC_ROWS, width), jnp.float32),
      pltpu.SemaphoreType.DMA,
  ]
  if width != 16:
    scratch.append(pltpu.VMEM_SHARED((N_NODES, width), jnp.float32))
    for _ in range(NBUF - 1):
      scratch.append(pltpu.VMEM((CH, width), jnp.float32))
      scratch.append(pltpu.SemaphoreType.DMA)

  @functools.partial(
      pl.kernel,
      mesh=_mesh,
      out_type=jax.ShapeDtypeStruct((NC, ACC_ROWS, width), jnp.float32),
      scratch_types=scratch,
      compiler_params=pltpu.CompilerParams(use_tc_tiling_on_sc=False),
  )
  def k(table_hbm, src_hbm, dst_hbm, zeros_hbm, out_hbm,
        src_v, dst_v, rows_v, acc_sh, sem, *maybe_tab):
    cid = lax.axis_index("c")
    sid = lax.axis_index("s")
    wid = cid * NS + sid
    sl = pl.ds(sid * RPT, RPT)
    startup = [
        pltpu.async_copy(zeros_hbm.at[sl], acc_sh.at[sl], sem),
        pltpu.async_copy(dst_hbm.at[wid], dst_v, sem),
    ]
    if width == 16:
      startup.append(pltpu.async_copy(table_hbm, rows_v, sem))
      for c in startup:
        c.wait()
      plsc.subcore_barrier()

      @pl.loop(0, NCHUNK)
      def _(j):
        pltpu.sync_copy(rows_v, acc_sh.at[dst_v.at[j]], add=True)
    else:
      tab_sh = maybe_tab[0]
      bufs = [rows_v] + list(maybe_tab[1::2])
      sems = [sem] + list(maybe_tab[2::2])
      tsl = pl.ds(sid * TPT, TPT)
      startup.append(pltpu.async_copy(src_hbm.at[wid], src_v, sem))
      startup.append(pltpu.async_copy(table_hbm.at[tsl], tab_sh.at[tsl], sem))
      for c in startup:
        c.wait()
      plsc.subcore_barrier()

      for b in range(NBUF - 1):
        pltpu.async_copy(tab_sh.at[src_v.at[b]], bufs[b], sems[b])

      @pl.loop(0, NCHUNK // NBUF)
      def _(i):
        j = i * NBUF
        for b in range(NBUF):
          jb = j + b
          nxt = jb + NBUF - 1
          nb = (b + NBUF - 1) % NBUF

          @pl.when(nxt < NCHUNK)
          def _(nxt=nxt, nb=nb):
            pltpu.async_copy(tab_sh.at[src_v.at[nxt]], bufs[nb], sems[nb])

          pltpu.make_async_copy(
              tab_sh.at[src_v.at[jb]], bufs[b], sems[b]).wait()
          pltpu.sync_copy(bufs[b], acc_sh.at[dst_v.at[jb]], add=True)

    plsc.subcore_barrier()
    pltpu.sync_copy(acc_sh.at[sl], out_hbm.at[cid, sl])

  return k


_sc_degree = _sc_scatter_kernel(16)
_sc_aggregate = _sc_scatter_kernel(64)


def _tc_table1(deg_parts, x, w):
  def body(p_ref, x_ref, w_ref, dinv_ref, tab_ref):
    deg = p_ref[0, :N_NODES, 0:1] + p_ref[1, :N_NODES, 0:1] + 1.0
    dinv = lax.rsqrt(deg)
    dinv_ref[...] = dinv
    xw = jnp.dot(x_ref[...], w_ref[...],
                 preferred_element_type=jnp.float32,
                 precision=lax.Precision.HIGHEST)
    tab_ref[...] = xw * dinv

  return pl.pallas_call(
      body,
      out_shape=(
          jax.ShapeDtypeStruct((N_NODES, 1), jnp.float32),
          jax.ShapeDtypeStruct((N_NODES, HIDDEN), jnp.float32),
      ),
  )(deg_parts, x, w)


def _tc_mid(parts, tab1, dinv, b1, w2):
  def body(p_ref, t_ref, d_ref, b_ref, w_ref, o_ref):
    s = p_ref[0, :N_NODES] + p_ref[1, :N_NODES]
    dinv = d_ref[...]
    h = jnp.maximum((s + t_ref[...]) * dinv + b_ref[...], 0.0)
    o_ref[...] = jnp.dot(h, w_ref[...],
                         preferred_element_type=jnp.float32,
                         precision=lax.Precision.HIGHEST) * dinv

  return pl.pallas_call(
      body,
      out_shape=jax.ShapeDtypeStruct((N_NODES, NCLASS), jnp.float32),
  )(parts, tab1, dinv, b1, w2)


def _tc_out(parts, tab2, dinv, b2):
  def body(p_ref, t_ref, d_ref, b_ref, o_ref):
    s = p_ref[0, :N_NODES] + p_ref[1, :N_NODES]
    o_ref[...] = (s + t_ref[...]) * d_ref[...] + b_ref[...]

  return pl.pallas_call(
      body,
      out_shape=jax.ShapeDtypeStruct((N_NODES, NCLASS), jnp.float32),
  )(parts, tab2, dinv, b2)


def kernel(x, edge_index, W1, b1, W2, b2):
  src = edge_index[0].astype(jnp.int32)
  dst = edge_index[1].astype(jnp.int32)
  pad = E_PAD - N_EDGES
  src3 = jnp.concatenate([src, jnp.zeros((pad,), jnp.int32)]).reshape(
      NW, NCHUNK, CH)
  dst3 = jnp.concatenate([dst, jnp.full((pad,), N_NODES, jnp.int32)]).reshape(
      NW, NCHUNK, CH)

  zeros16 = jnp.zeros((ACC_ROWS, 16), jnp.float32)
  zeros64 = jnp.zeros((ACC_ROWS, 64), jnp.float32)
  ones16 = jnp.ones((CH, 16), jnp.float32)

  deg_parts = _sc_degree(ones16, src3, dst3, zeros16)
  dinv, tab1 = _tc_table1(deg_parts, x, W1)

  s1 = _sc_aggregate(tab1, src3, dst3, zeros64)
  tab2 = _tc_mid(s1, tab1, dinv, b1.reshape(1, HIDDEN), W2)

  s2 = _sc_aggregate(tab2, src3, dst3, zeros64)
  return _tc_out(s2, tab2, dinv, b2.reshape(1, NCLASS))


# speedup vs baseline: 2.2730x; 1.0086x over previous
import functools

import jax
import jax.numpy as jnp
from jax import lax
from jax.experimental import pallas as pl
from jax.experimental.pallas import tpu as pltpu
from jax.experimental.pallas import tpu_sc as plsc

N_NODES = 10000
NFEAT = 128
HIDDEN = 64
NCLASS = 64
N_EDGES = 320000

NC = 2
NS = 16
NW = NC * NS
CH = 128
NCHUNK = 80
NBUF = 2
E_PER_TILE = NCHUNK * CH
E_PAD = NW * E_PER_TILE
ACC_ROWS = 10240
RPT = ACC_ROWS // NS
TPT = N_NODES // NS

_mesh = plsc.VectorSubcoreMesh(core_axis_name="c", subcore_axis_name="s")


def _sc_scatter_kernel(width):
  scratch = [
      pltpu.VMEM((NCHUNK, CH), jnp.int32),
      pltpu.VMEM((NCHUNK, CH), jnp.int32),
      pltpu.VMEM((CH, width), jnp.float32),
      pltpu.VMEM_SHARED((ACC_ROWS, width), jnp.float32),
      pltpu.SemaphoreType.DMA,
  ]
  if width != 16:
    scratch.append(pltpu.VMEM_SHARED((N_NODES, width), jnp.float32))
    for _ in range(NBUF - 1):
      scratch.append(pltpu.VMEM((CH, width), jnp.float32))
      scratch.append(pltpu.SemaphoreType.DMA)
    for _ in range(NBUF):
      scratch.append(pltpu.SemaphoreType.DMA)

  @functools.partial(
      pl.kernel,
      mesh=_mesh,
      out_type=jax.ShapeDtypeStruct((NC, ACC_ROWS, width), jnp.float32),
      scratch_types=scratch,
      compiler_params=pltpu.CompilerParams(use_tc_tiling_on_sc=False),
  )
  def k(table_hbm, src_hbm, dst_hbm, zeros_hbm, out_hbm,
        src_v, dst_v, rows_v, acc_sh, sem, *maybe_tab):
    cid = lax.axis_index("c")
    sid = lax.axis_index("s")
    wid = cid * NS + sid
    sl = pl.ds(sid * RPT, RPT)
    startup = [
        pltpu.async_copy(zeros_hbm.at[sl], acc_sh.at[sl], sem),
        pltpu.async_copy(dst_hbm.at[wid], dst_v, sem),
    ]
    if width == 16:
      startup.append(pltpu.async_copy(table_hbm, rows_v, sem))
      for c in startup:
        c.wait()
      plsc.subcore_barrier()

      @pl.loop(0, NCHUNK)
      def _(j):
        pltpu.sync_copy(rows_v, acc_sh.at[dst_v.at[j]], add=True)
    else:
      tab_sh = maybe_tab[0]
      bufs = [rows_v] + list(maybe_tab[1:2 * NBUF - 2:2])
      sems = [sem] + list(maybe_tab[2:2 * NBUF - 1:2])
      ssems = list(maybe_tab[2 * NBUF - 1:])
      tsl = pl.ds(sid * TPT, TPT)
      startup.append(pltpu.async_copy(src_hbm.at[wid], src_v, sem))
      startup.append(pltpu.async_copy(table_hbm.at[tsl], tab_sh.at[tsl], sem))
      for c in startup:
        c.wait()
      plsc.subcore_barrier()

      pltpu.async_copy(tab_sh.at[src_v.at[0]], bufs[0], sems[0])

      @pl.loop(0, NCHUNK // NBUF)
      def _(i):
        j = i * NBUF
        for b in range(NBUF):
          jb = j + b
          nb = (b + 1) % NBUF
          pltpu.make_async_copy(
              tab_sh.at[src_v.at[jb]], bufs[b], sems[b]).wait()
          pltpu.async_copy(bufs[b], acc_sh.at[dst_v.at[jb]], ssems[b],
                           add=True)

          @pl.when(jb >= 1)
          def _(nb=nb, jb=jb):
            pltpu.make_async_copy(
                bufs[nb], acc_sh.at[dst_v.at[jb - 1]], ssems[nb]).wait()

          @pl.when(jb + 1 < NCHUNK)
          def _(nb=nb, jb=jb):
            pltpu.async_copy(tab_sh.at[src_v.at[jb + 1]], bufs[nb], sems[nb])

      pltpu.make_async_copy(
          bufs[(NCHUNK - 1) % NBUF], acc_sh.at[dst_v.at[NCHUNK - 1]],
          ssems[(NCHUNK - 1) % NBUF]).wait()

    plsc.subcore_barrier()
    pltpu.sync_copy(acc_sh.at[sl], out_hbm.at[cid, sl])

  return k


_sc_degree = _sc_scatter_kernel(16)
_sc_aggregate = _sc_scatter_kernel(64)


def _tc_table1(deg_parts, x, w):
  def body(p_ref, x_ref, w_ref, dinv_ref, tab_ref):
    deg = p_ref[0, :N_NODES, 0:1] + p_ref[1, :N_NODES, 0:1] + 1.0
    dinv = lax.rsqrt(deg)
    dinv_ref[...] = dinv
    xw = jnp.dot(x_ref[...], w_ref[...],
                 preferred_element_type=jnp.float32,
                 precision=lax.Precision.HIGHEST)
    tab_ref[...] = xw * dinv

  return pl.pallas_call(
      body,
      out_shape=(
          jax.ShapeDtypeStruct((N_NODES, 1), jnp.float32),
          jax.ShapeDtypeStruct((N_NODES, HIDDEN), jnp.float32),
      ),
  )(deg_parts, x, w)


def _tc_mid(parts, tab1, dinv, b1, w2):
  def body(p_ref, t_ref, d_ref, b_ref, w_ref, o_ref):
    s = p_ref[0, :N_NODES] + p_ref[1, :N_NODES]
    dinv = d_ref[...]
    h = jnp.maximum((s + t_ref[...]) * dinv + b_ref[...], 0.0)
    o_ref[...] = jnp.dot(h, w_ref[...],
                         preferred_element_type=jnp.float32,
                         precision=lax.Precision.HIGHEST) * dinv

  return pl.pallas_call(
      body,
      out_shape=jax.ShapeDtypeStruct((N_NODES, NCLASS), jnp.float32),
  )(parts, tab1, dinv, b1, w2)


def _tc_out(parts, tab2, dinv, b2):
  def body(p_ref, t_ref, d_ref, b_ref, o_ref):
    s = p_ref[0, :N_NODES] + p_ref[1, :N_NODES]
    o_ref[...] = (s + t_ref[...]) * d_ref[...] + b_ref[...]

  return pl.pallas_call(
      body,
      out_shape=jax.ShapeDtypeStruct((N_NODES, NCLASS), jnp.float32),
  )(parts, tab2, dinv, b2)


def kernel(x, edge_index, W1, b1, W2, b2):
  src = edge_index[0].astype(jnp.int32)
  dst = edge_index[1].astype(jnp.int32)
  pad = E_PAD - N_EDGES
  src3 = jnp.concatenate([src, jnp.zeros((pad,), jnp.int32)]).reshape(
      NW, NCHUNK, CH)
  dst3 = jnp.concatenate([dst, jnp.full((pad,), N_NODES, jnp.int32)]).reshape(
      NW, NCHUNK, CH)

  zeros16 = jnp.zeros((ACC_ROWS, 16), jnp.float32)
  zeros64 = jnp.zeros((ACC_ROWS, 64), jnp.float32)
  ones16 = jnp.ones((CH, 16), jnp.float32)

  deg_parts = _sc_degree(ones16, src3, dst3, zeros16)
  dinv, tab1 = _tc_table1(deg_parts, x, W1)

  s1 = _sc_aggregate(tab1, src3, dst3, zeros64)
  tab2 = _tc_mid(s1, tab1, dinv, b1.reshape(1, HIDDEN), W2)

  s2 = _sc_aggregate(tab2, src3, dst3, zeros64)
  return _tc_out(s2, tab2, dinv, b2.reshape(1, NCLASS))


# speedup vs baseline: 2.2749x; 1.0008x over previous
import functools

import jax
import jax.numpy as jnp
from jax import lax
from jax.experimental import pallas as pl
from jax.experimental.pallas import tpu as pltpu
from jax.experimental.pallas import tpu_sc as plsc

N_NODES = 10000
NFEAT = 128
HIDDEN = 64
NCLASS = 64
N_EDGES = 320000

NC = 2
NS = 16
NW = NC * NS
CH = 128
NCHUNK = 80
NBUF = 2
E_PER_TILE = NCHUNK * CH
E_PAD = NW * E_PER_TILE
ACC_ROWS = 10240
RPT = ACC_ROWS // NS
TPT = N_NODES // NS

_mesh = plsc.VectorSubcoreMesh(core_axis_name="c", subcore_axis_name="s")


def _sc_scatter_kernel(width):
  scratch = [
      pltpu.VMEM((NCHUNK, CH), jnp.int32),
      pltpu.VMEM((NCHUNK, CH), jnp.int32),
      pltpu.VMEM((CH, width), jnp.float32),
      pltpu.VMEM_SHARED((ACC_ROWS, width), jnp.float32),
      pltpu.SemaphoreType.DMA,
  ]
  if width != 16:
    scratch.append(pltpu.VMEM_SHARED((N_NODES, width), jnp.float32))
    for _ in range(NBUF - 1):
      scratch.append(pltpu.VMEM((CH, width), jnp.float32))
      scratch.append(pltpu.SemaphoreType.DMA)
    for _ in range(NBUF):
      scratch.append(pltpu.SemaphoreType.DMA)

  @functools.partial(
      pl.kernel,
      mesh=_mesh,
      out_type=jax.ShapeDtypeStruct((NC, ACC_ROWS, width), jnp.float32),
      scratch_types=scratch,
      compiler_params=pltpu.CompilerParams(use_tc_tiling_on_sc=False),
  )
  def k(table_hbm, src_hbm, dst_hbm, zeros_hbm, out_hbm,
        src_v, dst_v, rows_v, acc_sh, sem, *maybe_tab):
    cid = lax.axis_index("c")
    sid = lax.axis_index("s")
    wid = cid * NS + sid
    sl = pl.ds(sid * RPT, RPT)
    startup = [
        pltpu.async_copy(zeros_hbm.at[sl], acc_sh.at[sl], sem),
        pltpu.async_copy(dst_hbm.at[wid], dst_v, sem),
    ]
    if width == 16:
      startup.append(pltpu.async_copy(table_hbm, rows_v, sem))
      for c in startup:
        c.wait()
      plsc.subcore_barrier()

      @pl.loop(0, NCHUNK)
      def _(j):
        pltpu.sync_copy(rows_v, acc_sh.at[dst_v.at[j]], add=True)
    else:
      tab_sh = maybe_tab[0]
      bufs = [rows_v] + list(maybe_tab[1:2 * NBUF - 2:2])
      sems = [sem] + list(maybe_tab[2:2 * NBUF - 1:2])
      ssems = list(maybe_tab[2 * NBUF - 1:])
      tsl = pl.ds(sid * TPT, TPT)
      startup.append(pltpu.async_copy(src_hbm.at[wid], src_v, sem))
      startup.append(pltpu.async_copy(table_hbm.at[tsl], tab_sh.at[tsl], sem))
      for c in startup:
        c.wait()
      plsc.subcore_barrier()

      for b in range(NBUF - 1):
        pltpu.async_copy(tab_sh.at[src_v.at[b]], bufs[b], sems[b])

      @pl.loop(0, NCHUNK // NBUF)
      def _(i):
        j = i * NBUF
        for b in range(NBUF):
          jb = j + b
          pb = (b + NBUF - 1) % NBUF
          pltpu.make_async_copy(
              tab_sh.at[src_v.at[jb]], bufs[b], sems[b]).wait()
          pltpu.async_copy(bufs[b], acc_sh.at[dst_v.at[jb]], ssems[b],
                           add=True)

          @pl.when(jb >= 1)
          def _(pb=pb, jb=jb):
            pltpu.make_async_copy(
                bufs[pb], acc_sh.at[dst_v.at[jb - 1]], ssems[pb]).wait()

          @pl.when(jb + NBUF - 1 < NCHUNK)
          def _(pb=pb, jb=jb):
            pltpu.async_copy(
                tab_sh.at[src_v.at[jb + NBUF - 1]], bufs[pb], sems[pb])

      pltpu.make_async_copy(
          bufs[(NCHUNK - 1) % NBUF], acc_sh.at[dst_v.at[NCHUNK - 1]],
          ssems[(NCHUNK - 1) % NBUF]).wait()

    plsc.subcore_barrier()
    pltpu.sync_copy(acc_sh.at[sl], out_hbm.at[cid, sl])

  return k


_sc_degree = _sc_scatter_kernel(16)
_sc_aggregate = _sc_scatter_kernel(64)


def _tc_table1(deg_parts, x, w):
  def body(p_ref, x_ref, w_ref, dinv_ref, tab_ref):
    deg = p_ref[0, :N_NODES, 0:1] + p_ref[1, :N_NODES, 0:1] + 1.0
    dinv = lax.rsqrt(deg)
    dinv_ref[...] = dinv
    xw = jnp.dot(x_ref[...], w_ref[...],
                 preferred_element_type=jnp.float32,
                 precision=lax.Precision.HIGHEST)
    tab_ref[...] = xw * dinv

  return pl.pallas_call(
      body,
      out_shape=(
          jax.ShapeDtypeStruct((N_NODES, 1), jnp.float32),
          jax.ShapeDtypeStruct((N_NODES, HIDDEN), jnp.float32),
      ),
  )(deg_parts, x, w)


def _tc_mid(parts, tab1, dinv, b1, w2):
  def body(p_ref, t_ref, d_ref, b_ref, w_ref, o_ref):
    s = p_ref[0, :N_NODES] + p_ref[1, :N_NODES]
    dinv = d_ref[...]
    h = jnp.maximum((s + t_ref[...]) * dinv + b_ref[...], 0.0)
    o_ref[...] = jnp.dot(h, w_ref[...],
                         preferred_element_type=jnp.float32,
                         precision=lax.Precision.HIGHEST) * dinv

  return pl.pallas_call(
      body,
      out_shape=jax.ShapeDtypeStruct((N_NODES, NCLASS), jnp.float32),
  )(parts, tab1, dinv, b1, w2)


def _tc_out(parts, tab2, dinv, b2):
  def body(p_ref, t_ref, d_ref, b_ref, o_ref):
    s = p_ref[0, :N_NODES] + p_ref[1, :N_NODES]
    o_ref[...] = (s + t_ref[...]) * d_ref[...] + b_ref[...]

  return pl.pallas_call(
      body,
      out_shape=jax.ShapeDtypeStruct((N_NODES, NCLASS), jnp.float32),
  )(parts, tab2, dinv, b2)


def kernel(x, edge_index, W1, b1, W2, b2):
  src = edge_index[0].astype(jnp.int32)
  dst = edge_index[1].astype(jnp.int32)
  pad = E_PAD - N_EDGES
  src3 = jnp.concatenate([src, jnp.zeros((pad,), jnp.int32)]).reshape(
      NW, NCHUNK, CH)
  dst3 = jnp.concatenate([dst, jnp.full((pad,), N_NODES, jnp.int32)]).reshape(
      NW, NCHUNK, CH)

  zeros16 = jnp.zeros((ACC_ROWS, 16), jnp.float32)
  zeros64 = jnp.zeros((ACC_ROWS, 64), jnp.float32)
  ones16 = jnp.ones((CH, 16), jnp.float32)

  deg_parts = _sc_degree(ones16, src3, dst3, zeros16)
  dinv, tab1 = _tc_table1(deg_parts, x, W1)

  s1 = _sc_aggregate(tab1, src3, dst3, zeros64)
  tab2 = _tc_mid(s1, tab1, dinv, b1.reshape(1, HIDDEN), W2)

  s2 = _sc_aggregate(tab2, src3, dst3, zeros64)
  return _tc_out(s2, tab2, dinv, b2.reshape(1, NCLASS))


# speedup vs baseline: 2.4056x; 1.0575x over previous
import functools

import jax
import jax.numpy as jnp
from jax import lax
from jax.experimental import pallas as pl
from jax.experimental.pallas import tpu as pltpu
from jax.experimental.pallas import tpu_sc as plsc

N_NODES = 10000
NFEAT = 128
HIDDEN = 64
NCLASS = 64
N_EDGES = 320000

NC = 2
NS = 16
NW = NC * NS
CH = 128
NCHUNK = 81
NBUF = 3
E_PER_TILE = NCHUNK * CH
E_PAD = NW * E_PER_TILE
ACC_ROWS = 10240
RPT = ACC_ROWS // NS
TPT = N_NODES // NS

_mesh = plsc.VectorSubcoreMesh(core_axis_name="c", subcore_axis_name="s")


def _sc_scatter_kernel(width):
  scratch = [
      pltpu.VMEM((NCHUNK, CH), jnp.int32),
      pltpu.VMEM((NCHUNK, CH), jnp.int32),
      pltpu.VMEM((CH, width), jnp.float32),
      pltpu.VMEM_SHARED((ACC_ROWS, width), jnp.float32),
      pltpu.SemaphoreType.DMA,
  ]
  if width != 16:
    scratch.append(pltpu.VMEM_SHARED((N_NODES, width), jnp.float32))
    for _ in range(NBUF - 1):
      scratch.append(pltpu.VMEM((CH, width), jnp.float32))
      scratch.append(pltpu.SemaphoreType.DMA)
    for _ in range(NBUF):
      scratch.append(pltpu.SemaphoreType.DMA)

  @functools.partial(
      pl.kernel,
      mesh=_mesh,
      out_type=jax.ShapeDtypeStruct((NC, ACC_ROWS, width), jnp.float32),
      scratch_types=scratch,
      compiler_params=pltpu.CompilerParams(use_tc_tiling_on_sc=False),
  )
  def k(table_hbm, src_hbm, dst_hbm, zeros_hbm, out_hbm,
        src_v, dst_v, rows_v, acc_sh, sem, *maybe_tab):
    cid = lax.axis_index("c")
    sid = lax.axis_index("s")
    wid = cid * NS + sid
    sl = pl.ds(sid * RPT, RPT)
    startup = [
        pltpu.async_copy(zeros_hbm.at[sl], acc_sh.at[sl], sem),
        pltpu.async_copy(dst_hbm.at[wid], dst_v, sem),
    ]
    if width == 16:
      startup.append(pltpu.async_copy(table_hbm, rows_v, sem))
      for c in startup:
        c.wait()
      plsc.subcore_barrier()

      @pl.loop(0, NCHUNK)
      def _(j):
        pltpu.sync_copy(rows_v, acc_sh.at[dst_v.at[j]], add=True)
    else:
      tab_sh = maybe_tab[0]
      bufs = [rows_v] + list(maybe_tab[1:2 * NBUF - 2:2])
      sems = [sem] + list(maybe_tab[2:2 * NBUF - 1:2])
      ssems = list(maybe_tab[2 * NBUF - 1:])
      tsl = pl.ds(sid * TPT, TPT)
      startup.append(pltpu.async_copy(src_hbm.at[wid], src_v, sem))
      startup.append(pltpu.async_copy(table_hbm.at[tsl], tab_sh.at[tsl], sem))
      for c in startup:
        c.wait()
      plsc.subcore_barrier()

      for b in range(NBUF - 1):
        pltpu.async_copy(tab_sh.at[src_v.at[b]], bufs[b], sems[b])

      @pl.loop(0, NCHUNK // NBUF)
      def _(i):
        j = i * NBUF
        for b in range(NBUF):
          jb = j + b
          pb = (b + NBUF - 1) % NBUF
          pltpu.make_async_copy(
              tab_sh.at[src_v.at[jb]], bufs[b], sems[b]).wait()
          pltpu.async_copy(bufs[b], acc_sh.at[dst_v.at[jb]], ssems[b],
                           add=True)

          @pl.when(jb >= 1)
          def _(pb=pb, jb=jb):
            pltpu.make_async_copy(
                bufs[pb], acc_sh.at[dst_v.at[jb - 1]], ssems[pb]).wait()

          @pl.when(jb + NBUF - 1 < NCHUNK)
          def _(pb=pb, jb=jb):
            pltpu.async_copy(
                tab_sh.at[src_v.at[jb + NBUF - 1]], bufs[pb], sems[pb])

      pltpu.make_async_copy(
          bufs[(NCHUNK - 1) % NBUF], acc_sh.at[dst_v.at[NCHUNK - 1]],
          ssems[(NCHUNK - 1) % NBUF]).wait()

    plsc.subcore_barrier()
    pltpu.sync_copy(acc_sh.at[sl], out_hbm.at[cid, sl])

  return k


_sc_degree = _sc_scatter_kernel(16)
_sc_aggregate = _sc_scatter_kernel(64)


def _tc_table1(deg_parts, x, w):
  def body(p_ref, x_ref, w_ref, dinv_ref, tab_ref):
    deg = p_ref[0, :N_NODES, 0:1] + p_ref[1, :N_NODES, 0:1] + 1.0
    dinv = lax.rsqrt(deg)
    dinv_ref[...] = dinv
    xw = jnp.dot(x_ref[...], w_ref[...],
                 preferred_element_type=jnp.float32,
                 precision=lax.Precision.HIGHEST)
    tab_ref[...] = xw * dinv

  return pl.pallas_call(
      body,
      out_shape=(
          jax.ShapeDtypeStruct((N_NODES, 1), jnp.float32),
          jax.ShapeDtypeStruct((N_NODES, HIDDEN), jnp.float32),
      ),
  )(deg_parts, x, w)


def _tc_mid(parts, tab1, dinv, b1, w2):
  def body(p_ref, t_ref, d_ref, b_ref, w_ref, o_ref):
    s = p_ref[0, :N_NODES] + p_ref[1, :N_NODES]
    dinv = d_ref[...]
    h = jnp.maximum((s + t_ref[...]) * dinv + b_ref[...], 0.0)
    o_ref[...] = jnp.dot(h, w_ref[...],
                         preferred_element_type=jnp.float32,
                         precision=lax.Precision.HIGHEST) * dinv

  return pl.pallas_call(
      body,
      out_shape=jax.ShapeDtypeStruct((N_NODES, NCLASS), jnp.float32),
  )(parts, tab1, dinv, b1, w2)


def _tc_out(parts, tab2, dinv, b2):
  def body(p_ref, t_ref, d_ref, b_ref, o_ref):
    s = p_ref[0, :N_NODES] + p_ref[1, :N_NODES]
    o_ref[...] = (s + t_ref[...]) * d_ref[...] + b_ref[...]

  return pl.pallas_call(
      body,
      out_shape=jax.ShapeDtypeStruct((N_NODES, NCLASS), jnp.float32),
  )(parts, tab2, dinv, b2)


def kernel(x, edge_index, W1, b1, W2, b2):
  src = edge_index[0].astype(jnp.int32)
  dst = edge_index[1].astype(jnp.int32)
  pad = E_PAD - N_EDGES
  src3 = jnp.concatenate([src, jnp.zeros((pad,), jnp.int32)]).reshape(
      NW, NCHUNK, CH)
  dst3 = jnp.concatenate([dst, jnp.full((pad,), N_NODES, jnp.int32)]).reshape(
      NW, NCHUNK, CH)

  zeros16 = jnp.zeros((ACC_ROWS, 16), jnp.float32)
  zeros64 = jnp.zeros((ACC_ROWS, 64), jnp.float32)
  ones16 = jnp.ones((CH, 16), jnp.float32)

  deg_parts = _sc_degree(ones16, src3, dst3, zeros16)
  dinv, tab1 = _tc_table1(deg_parts, x, W1)

  s1 = _sc_aggregate(tab1, src3, dst3, zeros64)
  tab2 = _tc_mid(s1, tab1, dinv, b1.reshape(1, HIDDEN), W2)

  s2 = _sc_aggregate(tab2, src3, dst3, zeros64)
  return _tc_out(s2, tab2, dinv, b2.reshape(1, NCLASS))


# speedup vs baseline: 2.5224x; 1.0486x over previous
import functools

import jax
import jax.numpy as jnp
from jax import lax
from jax.experimental import pallas as pl
from jax.experimental.pallas import tpu as pltpu
from jax.experimental.pallas import tpu_sc as plsc

N_NODES = 10000
NFEAT = 128
HIDDEN = 64
NCLASS = 64
N_EDGES = 320000

NC = 2
NS = 16
NW = NC * NS
CH = 128
NCHUNK = 79
NBUF = 3
NFULL = (NCHUNK // NBUF) * NBUF
E_PER_TILE = NCHUNK * CH
E_PAD = NW * E_PER_TILE
ACC_ROWS = 10240
RPT = ACC_ROWS // NS
TPT = N_NODES // NS

_mesh = plsc.VectorSubcoreMesh(core_axis_name="c", subcore_axis_name="s")


def _sc_scatter_kernel(width):
  scratch = [
      pltpu.VMEM((NCHUNK, CH), jnp.int32),
      pltpu.VMEM((NCHUNK, CH), jnp.int32),
      pltpu.VMEM((CH, width), jnp.float32),
      pltpu.VMEM_SHARED((ACC_ROWS, width), jnp.float32),
      pltpu.SemaphoreType.DMA,
  ]
  if width != 16:
    scratch.append(pltpu.VMEM_SHARED((N_NODES, width), jnp.float32))
    for _ in range(NBUF - 1):
      scratch.append(pltpu.VMEM((CH, width), jnp.float32))
      scratch.append(pltpu.SemaphoreType.DMA)
    for _ in range(NBUF):
      scratch.append(pltpu.SemaphoreType.DMA)

  @functools.partial(
      pl.kernel,
      mesh=_mesh,
      out_type=jax.ShapeDtypeStruct((NC, ACC_ROWS, width), jnp.float32),
      scratch_types=scratch,
      compiler_params=pltpu.CompilerParams(use_tc_tiling_on_sc=False),
  )
  def k(table_hbm, src_hbm, dst_hbm, zeros_hbm, out_hbm,
        src_v, dst_v, rows_v, acc_sh, sem, *maybe_tab):
    cid = lax.axis_index("c")
    sid = lax.axis_index("s")
    wid = cid * NS + sid
    sl = pl.ds(sid * RPT, RPT)
    startup = [
        pltpu.async_copy(zeros_hbm.at[sl], acc_sh.at[sl], sem),
        pltpu.async_copy(dst_hbm.at[wid], dst_v, sem),
    ]
    if width == 16:
      startup.append(pltpu.async_copy(table_hbm, rows_v, sem))
      for c in startup:
        c.wait()
      plsc.subcore_barrier()

      @pl.loop(0, NCHUNK)
      def _(j):
        pltpu.sync_copy(rows_v, acc_sh.at[dst_v.at[j]], add=True)
    else:
      tab_sh = maybe_tab[0]
      bufs = [rows_v] + list(maybe_tab[1:2 * NBUF - 2:2])
      sems = [sem] + list(maybe_tab[2:2 * NBUF - 1:2])
      ssems = list(maybe_tab[2 * NBUF - 1:])
      tsl = pl.ds(sid * TPT, TPT)
      startup.append(pltpu.async_copy(src_hbm.at[wid], src_v, sem))
      startup.append(pltpu.async_copy(table_hbm.at[tsl], tab_sh.at[tsl], sem))
      for c in startup:
        c.wait()
      plsc.subcore_barrier()

      for b in range(NBUF - 1):
        pltpu.async_copy(tab_sh.at[src_v.at[b]], bufs[b], sems[b])

      @pl.loop(0, NFULL // NBUF)
      def _(i):
        j = i * NBUF
        for b in range(NBUF):
          jb = j + b
          pb = (b + NBUF - 1) % NBUF
          pltpu.make_async_copy(
              tab_sh.at[src_v.at[jb]], bufs[b], sems[b]).wait()
          pltpu.async_copy(bufs[b], acc_sh.at[dst_v.at[jb]], ssems[b],
                           add=True)

          @pl.when(jb >= 1)
          def _(pb=pb, jb=jb):
            pltpu.make_async_copy(
                bufs[pb], acc_sh.at[dst_v.at[jb - 1]], ssems[pb]).wait()

          @pl.when(jb + NBUF - 1 < NCHUNK)
          def _(pb=pb, jb=jb):
            pltpu.async_copy(
                tab_sh.at[src_v.at[jb + NBUF - 1]], bufs[pb], sems[pb])

      for jb in range(NFULL, NCHUNK):
        b = jb % NBUF
        pltpu.make_async_copy(
            tab_sh.at[src_v.at[jb]], bufs[b], sems[b]).wait()
        pltpu.async_copy(bufs[b], acc_sh.at[dst_v.at[jb]], ssems[b],
                         add=True)
        pb = (jb - 1) % NBUF
        pltpu.make_async_copy(
            bufs[pb], acc_sh.at[dst_v.at[jb - 1]], ssems[pb]).wait()

      pltpu.make_async_copy(
          bufs[(NCHUNK - 1) % NBUF], acc_sh.at[dst_v.at[NCHUNK - 1]],
          ssems[(NCHUNK - 1) % NBUF]).wait()

    plsc.subcore_barrier()
    pltpu.sync_copy(acc_sh.at[sl], out_hbm.at[cid, sl])

  return k


_sc_degree = _sc_scatter_kernel(16)
_sc_aggregate = _sc_scatter_kernel(64)


def _tc_table1(deg_parts, x, w):
  def body(p_ref, x_ref, w_ref, dinv_ref, tab_ref):
    deg = p_ref[0, :N_NODES, 0:1] + p_ref[1, :N_NODES, 0:1] + 1.0
    dinv = lax.rsqrt(deg)
    dinv_ref[...] = dinv
    xw = jnp.dot(x_ref[...], w_ref[...],
                 preferred_element_type=jnp.float32,
                 precision=lax.Precision.HIGHEST)
    tab_ref[...] = xw * dinv

  return pl.pallas_call(
      body,
      out_shape=(
          jax.ShapeDtypeStruct((N_NODES, 1), jnp.float32),
          jax.ShapeDtypeStruct((N_NODES, HIDDEN), jnp.float32),
      ),
  )(deg_parts, x, w)


def _tc_mid(parts, tab1, dinv, b1, w2):
  def body(p_ref, t_ref, d_ref, b_ref, w_ref, o_ref):
    s = p_ref[0, :N_NODES] + p_ref[1, :N_NODES]
    dinv = d_ref[...]
    h = jnp.maximum((s + t_ref[...]) * dinv + b_ref[...], 0.0)
    o_ref[...] = jnp.dot(h, w_ref[...],
                         preferred_element_type=jnp.float32,
                         precision=lax.Precision.HIGHEST) * dinv

  return pl.pallas_call(
      body,
      out_shape=jax.ShapeDtypeStruct((N_NODES, NCLASS), jnp.float32),
  )(parts, tab1, dinv, b1, w2)


def _tc_out(parts, tab2, dinv, b2):
  def body(p_ref, t_ref, d_ref, b_ref, o_ref):
    s = p_ref[0, :N_NODES] + p_ref[1, :N_NODES]
    o_ref[...] = (s + t_ref[...]) * d_ref[...] + b_ref[...]

  return pl.pallas_call(
      body,
      out_shape=jax.ShapeDtypeStruct((N_NODES, NCLASS), jnp.float32),
  )(parts, tab2, dinv, b2)


def kernel(x, edge_index, W1, b1, W2, b2):
  src = edge_index[0].astype(jnp.int32)
  dst = edge_index[1].astype(jnp.int32)
  pad = E_PAD - N_EDGES
  src3 = jnp.concatenate([src, jnp.zeros((pad,), jnp.int32)]).reshape(
      NW, NCHUNK, CH)
  dst3 = jnp.concatenate([dst, jnp.full((pad,), N_NODES, jnp.int32)]).reshape(
      NW, NCHUNK, CH)

  zeros16 = jnp.zeros((ACC_ROWS, 16), jnp.float32)
  zeros64 = jnp.zeros((ACC_ROWS, 64), jnp.float32)
  ones16 = jnp.ones((CH, 16), jnp.float32)

  deg_parts = _sc_degree(ones16, src3, dst3, zeros16)
  dinv, tab1 = _tc_table1(deg_parts, x, W1)

  s1 = _sc_aggregate(tab1, src3, dst3, zeros64)
  tab2 = _tc_mid(s1, tab1, dinv, b1.reshape(1, HIDDEN), W2)

  s2 = _sc_aggregate(tab2, src3, dst3, zeros64)
  return _tc_out(s2, tab2, dinv, b2.reshape(1, NCLASS))
